# baseline (speedup 1.0000x reference)
import functools

import jax
import jax.numpy as jnp
from jax import lax
from jax.lax import Precision as _Prec
from jax.experimental import pallas as pl
from jax.experimental.pallas import tpu as pltpu
from jax.experimental.pallas import tpu_sc as plsc

IC = 200
STEPS = 3
NSEG = 16
NTOK = 32768
T = 2048
NC = NTOK // T

_SC_NCORES = 2
_SC_NSUB = 16
NW = _SC_NCORES * _SC_NSUB
TPW = NTOK // NW
CHUNK = 128
NCHUNK = TPW // CHUNK
ROWW = 208
NV = IC // 16
ACCW = NSEG * ROWW


def _sc_body(x_hbm, seg_hbm, cos_hbm, out_hbm, rowbuf, segb, cosb, acc, sem):
    wid = lax.axis_index("s") * _SC_NCORES + lax.axis_index("c")
    base = wid * TPW

    zero = jnp.zeros((16,), jnp.float32)

    def zero_step(i, carry):
        acc[pl.ds(i * 16, 16)] = zero
        return carry
    lax.fori_loop(0, ACCW // 16, zero_step, 0)

    def chunk_step(k, carry):
        start = base + k * CHUNK
        pltpu.sync_copy(x_hbm.at[pl.ds(start * IC, CHUNK * IC)],
                        rowbuf.at[pl.ds(0, CHUNK * IC)])
        pltpu.sync_copy(seg_hbm.at[pl.ds(start, CHUNK)], segb)
        pltpu.sync_copy(cos_hbm.at[pl.ds(start, CHUNK)], cosb)

        def group_step(g, c2):
            seg_vec = segb[pl.ds(g * 16, 16)]
            cos_vec = cosb[pl.ds(g * 16, 16)]
            for l in range(16):
                seg_t = seg_vec[l]
                cos_t = cos_vec[l]
                off = (g * 16 + l) * IC
                abase = seg_t * ROWW
                for j in range(NV + 1):
                    v = rowbuf[pl.ds(off + 16 * j, 16)] * cos_t
                    plsc.addupdate(acc.at[pl.ds(abase + 16 * j, 16)], v)
            return c2
        lax.fori_loop(0, CHUNK // 16, group_step, 0)
        return carry
    lax.fori_loop(0, NCHUNK, chunk_step, 0)

    pltpu.sync_copy(acc, out_hbm.at[wid])


def _sc_asit(x_flat, seg, cos):
    mesh = plsc.VectorSubcoreMesh(core_axis_name="c", subcore_axis_name="s")
    k = functools.partial(
        pl.kernel, mesh=mesh,
        out_type=jax.ShapeDtypeStruct((NW, ACCW), jnp.float32),
        scratch_types=[
            pltpu.VMEM((CHUNK * IC + 16,), jnp.float32),
            pltpu.VMEM((CHUNK,), jnp.int32),
            pltpu.VMEM((CHUNK,), jnp.float32),
            pltpu.VMEM((ACCW,), jnp.float32),
            pltpu.SemaphoreType.DMA,
        ],
    )(_sc_body)
    return k(x_flat, seg, cos)


def _sigmoid(z):
    return 1.0 / (1.0 + jnp.exp(-z))


def _tanh(z):
    return 1.0 - 2.0 / (jnp.exp(2.0 * z) + 1.0)


def _tc_body(x_ref, segr_ref, part_ref, qstar_ref, wihT_ref, whhT_ref,
             bih_ref, bhh_ref, out_ref, e_ref):
    f32 = jnp.float32

    def seg_mask(c):
        seg = segr_ref[:, pl.ds(c * T, T)]
        return jax.lax.broadcasted_iota(jnp.int32, (NSEG, T), 0) == seg

    def x_chunk(c):
        return x_ref[pl.ds(c * T, T), :]

    rows = jax.lax.broadcasted_iota(jnp.int32, (NSEG, NW * NSEG), 1)
    sel = (rows % NSEG ==
           jax.lax.broadcasted_iota(jnp.int32, (NSEG, NW * NSEG), 0))
    a_sit = jnp.dot(sel.astype(f32), part_ref[...],
                    preferred_element_type=f32,
                    precision=_Prec.HIGHEST)[:, :IC]

    h = a_sit
    c_st = jnp.zeros((NSEG, IC), f32)
    q_star = qstar_ref[...]
    wihT = wihT_ref[...]
    whhT = whhT_ref[...]
    bih = bih_ref[...]
    bhh = bhh_ref[...]

    for _ in range(STEPS):
        gates = (jnp.dot(q_star, wihT, preferred_element_type=f32)
                 + bih
                 + jnp.dot(h, whhT, preferred_element_type=f32)
                 + bhh)
        i_g = _sigmoid(gates[:, 0 * IC:1 * IC])
        f_g = _sigmoid(gates[:, 1 * IC:2 * IC])
        g_g = _tanh(gates[:, 2 * IC:3 * IC])
        o_g = _sigmoid(gates[:, 3 * IC:4 * IC])
        c_st = f_g * c_st + i_g * g_g
        h = o_g * _tanh(c_st)
        q = h
        qT = jnp.swapaxes(q, 0, 1)

        def logits_step(c, m):
            ec = jnp.swapaxes(
                jnp.dot(x_chunk(c), qT, preferred_element_type=f32),
                0, 1)
            e_ref[:, pl.ds(c * T, T)] = ec
            mc = jnp.max(jnp.where(seg_mask(c), ec, -jnp.inf), axis=1,
                         keepdims=True)
            return jnp.maximum(m, mc)
        m = lax.fori_loop(0, NC, logits_step,
                          jnp.full((NSEG, 1), -jnp.inf, f32))
        m = jnp.where(jnp.isfinite(m), m, 0.0)

        def pool_step(c, carry):
            racc, d = carry
            ec = e_ref[:, pl.ds(c * T, T)]
            pc = jnp.exp(jnp.where(seg_mask(c), ec - m, -jnp.inf))
            d = d + jnp.sum(pc, axis=1, keepdims=True)
            racc = racc + jnp.dot(pc, x_chunk(c), preferred_element_type=f32)
            return racc, d
        racc, d = lax.fori_loop(
            0, NC, pool_step,
            (jnp.zeros((NSEG, IC), f32), jnp.zeros((NSEG, 1), f32)))
        r = racc / (d + 1e-16)
        q_star = jnp.concatenate([q, r], axis=1)

    out_ref[...] = q_star


def _tc_run(x, segr, part, q_star, wihT, whhT, bih, bhh):
    return pl.pallas_call(
        _tc_body,
        out_shape=jax.ShapeDtypeStruct((NSEG, 2 * IC), jnp.float32),
        scratch_shapes=[pltpu.VMEM((NSEG, NTOK), jnp.float32)],
    )(x, segr, part, q_star, wihT, whhT, bih, bhh)


def kernel(x, segment_ids, cos_flat, q_star, W_ih, W_hh, b_ih, b_hh):
    seg = segment_ids.astype(jnp.int32)
    part = _sc_asit(x.reshape(-1), seg, cos_flat)
    part = part.reshape(NW * NSEG, ROWW)
    segr = seg.reshape(1, NTOK)
    wihT = W_ih.T
    whhT = W_hh.T
    bih = b_ih.reshape(1, 4 * IC)
    bhh = b_hh.reshape(1, 4 * IC)
    return _tc_run(x, segr, part, q_star, wihT, whhT, bih, bhh)


# speedup vs baseline: 1.2350x; 1.2350x over previous
import functools

import jax
import jax.numpy as jnp
from jax import lax
from jax.lax import Precision as _Prec
from jax.experimental import pallas as pl
from jax.experimental.pallas import tpu as pltpu
from jax.experimental.pallas import tpu_sc as plsc

IC = 200
STEPS = 3
NSEG = 16
NTOK = 32768
T = 2048
NC = NTOK // T

_SC_NCORES = 2
_SC_NSUB = 16
NW = _SC_NCORES * _SC_NSUB
TPW = NTOK // NW
CHUNK = 128
NCHUNK = TPW // CHUNK
ROWW = 208
NV = IC // 16
ACCW = NSEG * ROWW


def _sc_body(x_hbm, seg_hbm, cos_hbm, out_hbm, rowbuf, segb, cosb, acc, sem):
    wid = lax.axis_index("s") * _SC_NCORES + lax.axis_index("c")
    base = wid * TPW

    zero = jnp.zeros((16,), jnp.float32)

    def zero_step(i, carry):
        acc[pl.ds(i * 16, 16)] = zero
        return carry
    lax.fori_loop(0, ACCW // 16, zero_step, 0)

    def chunk_step(k, carry):
        start = base + k * CHUNK
        pltpu.sync_copy(x_hbm.at[pl.ds(start * IC, CHUNK * IC)],
                        rowbuf.at[pl.ds(0, CHUNK * IC)])
        pltpu.sync_copy(seg_hbm.at[pl.ds(start, CHUNK)], segb)
        pltpu.sync_copy(cos_hbm.at[pl.ds(start, CHUNK)], cosb)

        def group_step(g, c2):
            abase_prev, racc = c2
            seg_vec = segb[pl.ds(g * 16, 16)]
            cos_vec = cosb[pl.ds(g * 16, 16)]
            for l in range(16):
                seg_t = seg_vec[l]
                cos_t = cos_vec[l]
                off = (g * 16 + l) * IC
                abase = seg_t * ROWW
                changed = abase != abase_prev

                @pl.when(changed)
                def _flush():
                    for j in range(NV + 1):
                        plsc.addupdate(acc.at[pl.ds(abase_prev + 16 * j, 16)],
                                       racc[j])
                racc = [
                    jnp.where(changed,
                              rowbuf[pl.ds(off + 16 * j, 16)] * cos_t,
                              racc[j] + rowbuf[pl.ds(off + 16 * j, 16)] * cos_t)
                    for j in range(NV + 1)
                ]
                abase_prev = abase
            return abase_prev, racc
        return lax.fori_loop(0, CHUNK // 16, group_step, carry)

    zvecs = [jnp.zeros((16,), jnp.float32) for _ in range(NV + 1)]
    abase_last, racc = lax.fori_loop(0, NCHUNK, chunk_step,
                                     (jnp.int32(0), zvecs))
    for j in range(NV + 1):
        plsc.addupdate(acc.at[pl.ds(abase_last + 16 * j, 16)], racc[j])

    pltpu.sync_copy(acc, out_hbm.at[wid])


def _sc_asit(x_flat, seg, cos):
    mesh = plsc.VectorSubcoreMesh(core_axis_name="c", subcore_axis_name="s")
    k = functools.partial(
        pl.kernel, mesh=mesh,
        out_type=jax.ShapeDtypeStruct((NW, ACCW), jnp.float32),
        scratch_types=[
            pltpu.VMEM((CHUNK * IC + 16,), jnp.float32),
            pltpu.VMEM((CHUNK,), jnp.int32),
            pltpu.VMEM((CHUNK,), jnp.float32),
            pltpu.VMEM((ACCW,), jnp.float32),
            pltpu.SemaphoreType.DMA,
        ],
    )(_sc_body)
    return k(x_flat, seg, cos)


def _sigmoid(z):
    return 1.0 / (1.0 + jnp.exp(-z))


def _tanh(z):
    return 1.0 - 2.0 / (jnp.exp(2.0 * z) + 1.0)


def _tc_body(x_ref, segr_ref, part_ref, qstar_ref, wihT_ref, whhT_ref,
             bih_ref, bhh_ref, out_ref, e_ref):
    f32 = jnp.float32

    def seg_mask(c):
        seg = segr_ref[:, pl.ds(c * T, T)]
        return jax.lax.broadcasted_iota(jnp.int32, (NSEG, T), 0) == seg

    def x_chunk(c):
        return x_ref[pl.ds(c * T, T), :]

    rows = jax.lax.broadcasted_iota(jnp.int32, (NSEG, NW * NSEG), 1)
    sel = (rows % NSEG ==
           jax.lax.broadcasted_iota(jnp.int32, (NSEG, NW * NSEG), 0))
    a_sit = jnp.dot(sel.astype(f32), part_ref[...],
                    preferred_element_type=f32,
                    precision=_Prec.HIGHEST)[:, :IC]

    h = a_sit
    c_st = jnp.zeros((NSEG, IC), f32)
    q_star = qstar_ref[...]
    wihT = wihT_ref[...]
    whhT = whhT_ref[...]
    bih = bih_ref[...]
    bhh = bhh_ref[...]

    for _ in range(STEPS):
        gates = (jnp.dot(q_star, wihT, preferred_element_type=f32)
                 + bih
                 + jnp.dot(h, whhT, preferred_element_type=f32)
                 + bhh)
        i_g = _sigmoid(gates[:, 0 * IC:1 * IC])
        f_g = _sigmoid(gates[:, 1 * IC:2 * IC])
        g_g = _tanh(gates[:, 2 * IC:3 * IC])
        o_g = _sigmoid(gates[:, 3 * IC:4 * IC])
        c_st = f_g * c_st + i_g * g_g
        h = o_g * _tanh(c_st)
        q = h
        qT = jnp.swapaxes(q, 0, 1)

        def logits_step(c, m):
            ec = jnp.swapaxes(
                jnp.dot(x_chunk(c), qT, preferred_element_type=f32),
                0, 1)
            e_ref[:, pl.ds(c * T, T)] = ec
            mc = jnp.max(jnp.where(seg_mask(c), ec, -jnp.inf), axis=1,
                         keepdims=True)
            return jnp.maximum(m, mc)
        m = lax.fori_loop(0, NC, logits_step,
                          jnp.full((NSEG, 1), -jnp.inf, f32))
        m = jnp.where(jnp.isfinite(m), m, 0.0)

        def pool_step(c, carry):
            racc, d = carry
            ec = e_ref[:, pl.ds(c * T, T)]
            pc = jnp.exp(jnp.where(seg_mask(c), ec - m, -jnp.inf))
            d = d + jnp.sum(pc, axis=1, keepdims=True)
            racc = racc + jnp.dot(pc, x_chunk(c), preferred_element_type=f32)
            return racc, d
        racc, d = lax.fori_loop(
            0, NC, pool_step,
            (jnp.zeros((NSEG, IC), f32), jnp.zeros((NSEG, 1), f32)))
        r = racc / (d + 1e-16)
        q_star = jnp.concatenate([q, r], axis=1)

    out_ref[...] = q_star


def _tc_run(x, segr, part, q_star, wihT, whhT, bih, bhh):
    return pl.pallas_call(
        _tc_body,
        out_shape=jax.ShapeDtypeStruct((NSEG, 2 * IC), jnp.float32),
        scratch_shapes=[pltpu.VMEM((NSEG, NTOK), jnp.float32)],
    )(x, segr, part, q_star, wihT, whhT, bih, bhh)


def kernel(x, segment_ids, cos_flat, q_star, W_ih, W_hh, b_ih, b_hh):
    seg = segment_ids.astype(jnp.int32)
    part = _sc_asit(x.reshape(-1), seg, cos_flat)
    part = part.reshape(NW * NSEG, ROWW)
    segr = seg.reshape(1, NTOK)
    wihT = W_ih.T
    whhT = W_hh.T
    bih = b_ih.reshape(1, 4 * IC)
    bhh = b_hh.reshape(1, 4 * IC)
    return _tc_run(x, segr, part, q_star, wihT, whhT, bih, bhh)


# speedup vs baseline: 2.2190x; 1.7967x over previous
import jax
import jax.numpy as jnp
from jax import lax
from jax.lax import Precision as _Prec
from jax.experimental import pallas as pl
from jax.experimental.pallas import tpu as pltpu

IC = 200
STEPS = 3
NSEG = 16
NTOK = 32768
T = 2048
NC = NTOK // T


def _sigmoid(z):
    return 1.0 / (1.0 + jnp.exp(-z))


def _tanh(z):
    return 1.0 - 2.0 / (jnp.exp(2.0 * z) + 1.0)


def _body(x_ref, segr_ref, cosr_ref, qstar_ref, wihT_ref, whhT_ref,
          bih_ref, bhh_ref, out_ref, e_ref):
    f32 = jnp.float32

    def seg_mask(c):
        seg = segr_ref[:, pl.ds(c * T, T)]
        return jax.lax.broadcasted_iota(jnp.int32, (NSEG, T), 0) == seg

    def x_chunk(c):
        return x_ref[pl.ds(c * T, T), :]

    def asit_step(c, acc):
        w = seg_mask(c).astype(f32) * cosr_ref[:, pl.ds(c * T, T)]
        return acc + jnp.dot(w, x_chunk(c), preferred_element_type=f32,
                             precision=_Prec.HIGHEST)
    a_sit = lax.fori_loop(0, NC, asit_step, jnp.zeros((NSEG, IC), f32))

    h = a_sit
    c_st = jnp.zeros((NSEG, IC), f32)
    q_star = qstar_ref[...]
    wihT = wihT_ref[...]
    whhT = whhT_ref[...]
    bih = bih_ref[...]
    bhh = bhh_ref[...]

    for _ in range(STEPS):
        gates = (jnp.dot(q_star, wihT, preferred_element_type=f32)
                 + bih
                 + jnp.dot(h, whhT, preferred_element_type=f32)
                 + bhh)
        i_g = _sigmoid(gates[:, 0 * IC:1 * IC])
        f_g = _sigmoid(gates[:, 1 * IC:2 * IC])
        g_g = _tanh(gates[:, 2 * IC:3 * IC])
        o_g = _sigmoid(gates[:, 3 * IC:4 * IC])
        c_st = f_g * c_st + i_g * g_g
        h = o_g * _tanh(c_st)
        q = h
        qT = jnp.swapaxes(q, 0, 1)

        def logits_step(c, m):
            ec = jnp.swapaxes(
                jnp.dot(x_chunk(c), qT, preferred_element_type=f32),
                0, 1)
            e_ref[:, pl.ds(c * T, T)] = ec
            mc = jnp.max(jnp.where(seg_mask(c), ec, -jnp.inf), axis=1,
                         keepdims=True)
            return jnp.maximum(m, mc)
        m = lax.fori_loop(0, NC, logits_step,
                          jnp.full((NSEG, 1), -jnp.inf, f32))
        m = jnp.where(jnp.isfinite(m), m, 0.0)

        def pool_step(c, carry):
            racc, d = carry
            ec = e_ref[:, pl.ds(c * T, T)]
            pc = jnp.exp(jnp.where(seg_mask(c), ec - m, -jnp.inf))
            d = d + jnp.sum(pc, axis=1, keepdims=True)
            racc = racc + jnp.dot(pc, x_chunk(c), preferred_element_type=f32)
            return racc, d
        racc, d = lax.fori_loop(
            0, NC, pool_step,
            (jnp.zeros((NSEG, IC), f32), jnp.zeros((NSEG, 1), f32)))
        r = racc / (d + 1e-16)
        q_star = jnp.concatenate([q, r], axis=1)

    out_ref[...] = q_star


def _run(x, segr, cosr, q_star, wihT, whhT, bih, bhh):
    return pl.pallas_call(
        _body,
        out_shape=jax.ShapeDtypeStruct((NSEG, 2 * IC), jnp.float32),
        scratch_shapes=[pltpu.VMEM((NSEG, NTOK), jnp.float32)],
    )(x, segr, cosr, q_star, wihT, whhT, bih, bhh)


def kernel(x, segment_ids, cos_flat, q_star, W_ih, W_hh, b_ih, b_hh):
    segr = segment_ids.astype(jnp.int32).reshape(1, NTOK)
    cosr = cos_flat.reshape(1, NTOK)
    wihT = W_ih.T
    whhT = W_hh.T
    bih = b_ih.reshape(1, 4 * IC)
    bhh = b_hh.reshape(1, 4 * IC)
    return _run(x, segr, cosr, q_star, wihT, whhT, bih, bhh)
